# baseline (speedup 1.0000x reference)
import functools
import jax
import jax.numpy as jnp
from jax import lax
from jax.experimental import pallas as pl
from jax.experimental.pallas import tpu as pltpu
from jax.experimental.pallas import tpu_sc as plsc

_D = 64
_L = 16
_NW = 32
_C = 512
_SCALE = 8.0


def _sc_embed(idx_flat, table):
    b_total = idx_flat.shape[0]
    b_per_w = b_total // _NW
    n_chunks = b_per_w // _C
    mesh = plsc.VectorSubcoreMesh(core_axis_name="c", subcore_axis_name="s")

    @functools.partial(
        pl.kernel,
        out_type=jax.ShapeDtypeStruct((b_total, _D), jnp.float32),
        mesh=mesh,
        scratch_types=[
            pltpu.VMEM((_C,), jnp.int32),
            pltpu.VMEM((_C, _D), jnp.float32),
            pltpu.SemaphoreType.DMA,
        ],
        compiler_params=pltpu.CompilerParams(use_tc_tiling_on_sc=False),
    )
    def k(table_hbm, idx_hbm, out_hbm, idx_v, rows_v, sem):
        wid = lax.axis_index("s") * 2 + lax.axis_index("c")
        base = wid * b_per_w

        @pl.loop(0, n_chunks)
        def _chunk(g):
            row0 = base + g * _C
            pltpu.sync_copy(idx_hbm.at[pl.ds(row0, _C)], idx_v)
            pltpu.async_copy(table_hbm.at[idx_v], rows_v, sem).wait()

            @pl.loop(0, _C)
            def _scale(r):
                for c in range(0, _D, _L):
                    rows_v[r, pl.ds(c, _L)] = rows_v[r, pl.ds(c, _L)] * _SCALE

            pltpu.sync_copy(rows_v, out_hbm.at[pl.ds(row0, _C)])

    return k(table, idx_flat)


def kernel(input, table):
    b, h = input.shape
    idx_flat = input.reshape(b * h)
    out = _sc_embed(idx_flat, table)
    return out.reshape(b, h, _D)


# speedup vs baseline: 1.1341x; 1.1341x over previous
import functools
import jax
import jax.numpy as jnp
from jax import lax
from jax.experimental import pallas as pl
from jax.experimental.pallas import tpu as pltpu
from jax.experimental.pallas import tpu_sc as plsc

_D = 64
_L = 16
_NW = 32
_C = 800
_SCALE = 8.0


def _sc_embed(idx_flat, table):
    b_total = idx_flat.shape[0]
    b_per_w = b_total // _NW
    n_chunks = b_per_w // _C
    assert n_chunks % 2 == 0 and n_chunks * _C == b_per_w
    mesh = plsc.VectorSubcoreMesh(core_axis_name="c", subcore_axis_name="s")

    @functools.partial(
        pl.kernel,
        out_type=jax.ShapeDtypeStruct((b_total, _D), jnp.float32),
        mesh=mesh,
        scratch_types=[
            [pltpu.VMEM((_C,), jnp.int32) for _ in range(2)],
            [pltpu.VMEM((_C, _D), jnp.float32) for _ in range(2)],
            [pltpu.SemaphoreType.DMA for _ in range(2)],
            [pltpu.SemaphoreType.DMA for _ in range(2)],
        ],
        compiler_params=pltpu.CompilerParams(use_tc_tiling_on_sc=False),
    )
    def k(table_hbm, idx_hbm, out_hbm, idx_v, rows_v, gsem, osem):
        wid = lax.axis_index("s") * 2 + lax.axis_index("c")
        base = wid * b_per_w

        def start_gather(g, slot):
            row0 = base + g * _C
            pltpu.sync_copy(idx_hbm.at[pl.ds(row0, _C)], idx_v[slot])
            pltpu.async_copy(table_hbm.at[idx_v[slot]], rows_v[slot],
                             gsem[slot])

        start_gather(0, 0)

        @pl.loop(0, n_chunks, step=2)
        def _outer(g0):
            for b in range(2):
                g = g0 + b
                slot = b
                nslot = 1 - b

                @pl.when(g >= 1)
                def _():
                    pltpu.make_async_copy(
                        rows_v[nslot],
                        out_hbm.at[pl.ds(base + (g - 1) * _C, _C)],
                        osem[nslot]).wait()

                @pl.when(g + 1 < n_chunks)
                def _():
                    start_gather(g + 1, nslot)

                pltpu.make_async_copy(table_hbm.at[idx_v[slot]],
                                      rows_v[slot], gsem[slot]).wait()

                buf = rows_v[slot]

                @plsc.parallel_loop(0, _C, unroll=4)
                def _scale(r):
                    for c in range(0, _D, _L):
                        buf[r, pl.ds(c, _L)] = buf[r, pl.ds(c, _L)] * _SCALE

                pltpu.async_copy(buf, out_hbm.at[pl.ds(base + g * _C, _C)],
                                 osem[slot])

        pltpu.make_async_copy(
            rows_v[1], out_hbm.at[pl.ds(base + (n_chunks - 1) * _C, _C)],
            osem[1]).wait()

    return k(table, idx_flat)


def kernel(input, table):
    b, h = input.shape
    idx_flat = input.reshape(b * h)
    out = _sc_embed(idx_flat, table)
    return out.reshape(b, h, _D)
